# baseline (speedup 1.0000x reference)
import jax
import jax.numpy as jnp
from jax.experimental import pallas as pl
from jax.experimental.pallas import tpu as pltpu

_MAXP = 128
_NPOS = 2 * _MAXP + 1
_D = 32


def _make_body(S, BQ, grid):
    W = S * _D // 128
    U = (2 * S - 1) // 4 + 1

    def body(table_ref, out_ref, g_ref, sem):
        i = pl.program_id(0)

        @pl.when(i == 0)
        def _build_g():
            t = table_ref[:, :]
            t0 = t[0:1, :]
            t_last = t[_NPOS - 1:_NPOS, :]

            def clamped(lo_pad, hi_pad):
                return jnp.concatenate(
                    [jnp.broadcast_to(t0, (lo_pad, _D)), t,
                     jnp.broadcast_to(t_last, (hi_pad, _D))], axis=0)

            ts = jnp.concatenate(
                [clamped(4 - cc, 3 + cc) for cc in range(4)], axis=1)
            n_iota = jax.lax.broadcasted_iota(jnp.int32, (U, _NPOS + 7), 1)
            u_iota = jax.lax.broadcasted_iota(jnp.int32, (U, _NPOS + 7), 0)
            for p in range(4):
                n0 = jnp.clip(4 * u_iota + (p - (S - 1 - _MAXP) + 4), 0,
                              _NPOS + 3)
                onehot = (n_iota == n0).astype(jnp.float32)
                g_ref[p, :, :] = jax.lax.dot_general(
                    onehot, ts, (((1,), (0,)), ((), ())),
                    preferred_element_type=jnp.float32)

        for k in range(BQ):
            q = i * BQ + k
            phase = (S - 1 - k) % 4
            e = (S - 1 - q - phase) // 4
            pltpu.make_async_copy(
                g_ref.at[phase, pl.ds(e, W), :],
                out_ref.at[pl.ds(q * W, W)], sem).start()

        @pl.when(i > 0)
        def _wait_prev():
            for _ in range(BQ):
                pltpu.make_async_copy(
                    g_ref.at[0, pl.ds(0, W), :],
                    out_ref.at[pl.ds(0, W)], sem).wait()

        @pl.when(i == grid - 1)
        def _drain():
            for _ in range(BQ):
                pltpu.make_async_copy(
                    g_ref.at[0, pl.ds(0, W), :],
                    out_ref.at[pl.ds(0, W)], sem).wait()

    return body


def kernel(inputs, table):
    S = inputs.shape[1]
    BQ = 16
    grid = S // BQ
    W = S * _D // 128
    U = (2 * S - 1) // 4 + 1
    out = pl.pallas_call(
        _make_body(S, BQ, grid),
        grid=(grid,),
        in_specs=[pl.BlockSpec(memory_space=pltpu.MemorySpace.VMEM)],
        out_specs=pl.BlockSpec(memory_space=pl.ANY),
        out_shape=jax.ShapeDtypeStruct((S * W, 128), jnp.float32),
        scratch_shapes=[
            pltpu.VMEM((4, U, 128), jnp.float32),
            pltpu.SemaphoreType.DMA,
        ],
    )(table)
    return out.reshape(S, S, _D)


# speedup vs baseline: 1.4818x; 1.4818x over previous
import jax
import jax.numpy as jnp
from jax.experimental import pallas as pl
from jax.experimental.pallas import tpu as pltpu

_MAXP = 128
_NPOS = 2 * _MAXP + 1
_D = 32


def _make_body(S, q0, NQ, BQ, grid):
    W = S * _D // 128
    U = (2 * S - 1) // 4 + 1

    def body(table_ref, out_ref, g_ref, sem):
        i = pl.program_id(0)

        @pl.when(i == 0)
        def _build_g():
            t = table_ref[:, :]
            t0 = t[0:1, :]
            t_last = t[_NPOS - 1:_NPOS, :]

            def clamped(lo_pad, hi_pad):
                return jnp.concatenate(
                    [jnp.broadcast_to(t0, (lo_pad, _D)), t,
                     jnp.broadcast_to(t_last, (hi_pad, _D))], axis=0)

            ts = jnp.concatenate(
                [clamped(4 - cc, 3 + cc) for cc in range(4)], axis=1)
            n_iota = jax.lax.broadcasted_iota(jnp.int32, (U, _NPOS + 7), 1)
            u_iota = jax.lax.broadcasted_iota(jnp.int32, (U, _NPOS + 7), 0)
            for p in range(4):
                n0 = jnp.clip(4 * u_iota + (p - (S - 1 - _MAXP) + 4), 0,
                              _NPOS + 3)
                onehot = (n_iota == n0).astype(jnp.float32)
                g_ref[p, :, :] = jax.lax.dot_general(
                    onehot, ts, (((1,), (0,)), ((), ())),
                    preferred_element_type=jnp.float32)

        for k in range(BQ):
            q = q0 + i * BQ + k
            phase = (S - 1 - (q0 % 4) - k) % 4
            e = (S - 1 - q - phase) // 4
            pltpu.make_async_copy(
                g_ref.at[phase, pl.ds(e, W), :],
                out_ref.at[i * BQ + k], sem).start()

        @pl.when(i > 0)
        def _wait_prev():
            for _ in range(BQ):
                pltpu.make_async_copy(
                    g_ref.at[0, pl.ds(0, W), :], out_ref.at[0], sem).wait()

        @pl.when(i == grid - 1)
        def _drain():
            for _ in range(BQ):
                pltpu.make_async_copy(
                    g_ref.at[0, pl.ds(0, W), :], out_ref.at[0], sem).wait()

    return body


def _half(table, S, q0, NQ):
    BQ = 16
    grid = NQ // BQ
    W = S * _D // 128
    U = (2 * S - 1) // 4 + 1
    return pl.pallas_call(
        _make_body(S, q0, NQ, BQ, grid),
        grid=(grid,),
        in_specs=[pl.BlockSpec(memory_space=pltpu.MemorySpace.VMEM)],
        out_specs=pl.BlockSpec(memory_space=pl.ANY),
        out_shape=jax.ShapeDtypeStruct((NQ, W, 128), jnp.float32),
        scratch_shapes=[
            pltpu.VMEM((4, U, 128), jnp.float32),
            pltpu.SemaphoreType.DMA,
        ],
    )(table)


def kernel(inputs, table):
    S = inputs.shape[1]
    NQ = S // 2
    a = _half(table, S, 0, NQ).reshape(NQ, S, _D)
    b = _half(table, S, NQ, NQ).reshape(NQ, S, _D)
    return jnp.concatenate([a, b], axis=0)


# speedup vs baseline: 1.7740x; 1.1972x over previous
import functools
import jax
import jax.numpy as jnp
from jax import lax
from jax.experimental import pallas as pl
from jax.experimental.pallas import tpu as pltpu
from jax.experimental.pallas import tpu_sc as plsc

_MAXP = 128
_NPOS = 2 * _MAXP + 1
_D = 32
_NC = 2
_NS = 16


def _build_g(table, S):
    U = (2 * S - 1) // 4 + 1

    def body(table_ref, g_ref):
        t = table_ref[:, :]
        t0 = t[0:1, :]
        t_last = t[_NPOS - 1:_NPOS, :]

        def clamped(lo_pad, hi_pad):
            return jnp.concatenate(
                [jnp.broadcast_to(t0, (lo_pad, _D)), t,
                 jnp.broadcast_to(t_last, (hi_pad, _D))], axis=0)

        ts = jnp.concatenate(
            [clamped(4 - cc, 3 + cc) for cc in range(4)], axis=1)
        n_iota = jax.lax.broadcasted_iota(jnp.int32, (U, _NPOS + 7), 1)
        u_iota = jax.lax.broadcasted_iota(jnp.int32, (U, _NPOS + 7), 0)
        for p in range(4):
            n0 = jnp.clip(4 * u_iota + (p - (S - 1 - _MAXP) + 4), 0,
                          _NPOS + 3)
            onehot = (n_iota == n0).astype(jnp.float32)
            g_ref[p, :, :] = jax.lax.dot_general(
                onehot, ts, (((1,), (0,)), ((), ())),
                preferred_element_type=jnp.float32)

    return pl.pallas_call(
        body,
        out_shape=jax.ShapeDtypeStruct((4, U, 128), jnp.float32),
    )(table)


def _spread(g, S):
    W = S * _D // 128
    U = (2 * S - 1) // 4 + 1
    QW = S // (_NC * _NS)
    mesh = plsc.VectorSubcoreMesh(core_axis_name="c", subcore_axis_name="s")

    @functools.partial(
        pl.kernel,
        out_type=jax.ShapeDtypeStruct((S, W, 128), jnp.float32),
        mesh=mesh,
        scratch_types=[
            pltpu.VMEM_SHARED((4, U, 128), jnp.float32),
            pltpu.SemaphoreType.DMA,
        ],
    )
    def k(g_hbm, out_hbm, shared, sem):
        sid = lax.axis_index("s")

        @pl.when(sid == 0)
        def _load():
            pltpu.sync_copy(g_hbm, shared)

        plsc.subcore_barrier()

        wid = lax.axis_index("c") * _NS + sid
        base = wid * QW

        @pl.loop(0, QW // 4)
        def _fire(jj):
            for m in range(4):
                q = base + 4 * jj + m
                phase = (S - 1 - m) % 4
                e = (S - 1 - q - phase) // 4
                pltpu.make_async_copy(
                    shared.at[phase, pl.ds(e, W), :], out_hbm.at[q],
                    sem).start()

        @pl.loop(0, QW)
        def _drain(j):
            pltpu.make_async_copy(
                shared.at[0, pl.ds(0, W), :], out_hbm.at[0], sem).wait()

    return k(g)


def kernel(inputs, table):
    S = inputs.shape[1]
    g = _build_g(table, S)
    out = _spread(g, S)
    return out.reshape(S, S, _D)
